# Initial kernel scaffold; baseline (speedup 1.0000x reference)
import jax
import jax.numpy as jnp
from jax.experimental import pallas as pl


def kernel(x):
    raise NotImplementedError("write your pallas kernel here")


# speedup vs baseline: 4.2842x; 4.2842x over previous
import functools

import jax
import jax.numpy as jnp
from jax import lax
from jax.experimental import pallas as pl
from jax.experimental.pallas import tpu as pltpu
from jax.experimental.pallas import tpu_sc as plsc

B = 128
N = 32768
K = 1639
QA = N - K + 1
NB12 = 2048
NB3 = 1024
NW = 32
ROWS = B // NW
INT_MIN = -(2**31)


def _keys(xv):
    b = plsc.bitcast(xv, jnp.int32)
    t = b ^ ((b >> 31) | INT_MIN)
    return plsc.bitcast(t, jnp.uint32)


def _clear(hist, nwords):
    z = jnp.zeros((16,), jnp.int32)

    def body(i, _):
        hist[pl.ds(i * 16, 16)] = z
        return 0

    lax.fori_loop(0, nwords // 16, body, 0)


def _scan2(hist, nb, qa, qb):
    iota = lax.iota(jnp.int32, 16)

    def body(g, carry):
        run, bin_a, base_a, bin_b, base_b = carry
        tot = hist[pl.ds(g * 16, 16)]
        for l in range(1, 16):
            tot = tot + hist[pl.ds(l * nb + g * 16, 16)]
        cum = run + plsc.cumsum(tot)
        prev = cum - tot
        ids = g * 16 + iota
        ma = (prev < qa) & (cum >= qa)
        mb = (prev < qb) & (cum >= qb)
        bin_a = bin_a + jnp.sum(jnp.where(ma, ids, 0))
        base_a = base_a + jnp.sum(jnp.where(ma, prev, 0))
        bin_b = bin_b + jnp.sum(jnp.where(mb, ids, 0))
        base_b = base_b + jnp.sum(jnp.where(mb, prev, 0))
        return run + jnp.sum(tot), bin_a, base_a, bin_b, base_b

    z = jnp.int32(0)
    _, bin_a, base_a, bin_b, base_b = lax.fori_loop(
        0, nb // 16, body, (z, z, z, z, z))
    return bin_a, base_a, bin_b, base_b


def _scan1(hist, nb, q):
    iota = lax.iota(jnp.int32, 16)

    def body(g, carry):
        run, bin_, base = carry
        tot = hist[pl.ds(g * 16, 16)]
        for l in range(1, 16):
            tot = tot + hist[pl.ds(l * nb + g * 16, 16)]
        cum = run + plsc.cumsum(tot)
        prev = cum - tot
        m = (prev < q) & (cum >= q)
        bin_ = bin_ + jnp.sum(jnp.where(m, g * 16 + iota, 0))
        base = base + jnp.sum(jnp.where(m, prev, 0))
        return run + jnp.sum(tot), bin_, base

    z = jnp.int32(0)
    _, bin_, base = lax.fori_loop(0, nb // 16, body, (z, z, z))
    return bin_, base


def _inv_key(key_i32):
    bits = key_i32 ^ jnp.where(key_i32 < 0, INT_MIN, jnp.int32(-1))
    return plsc.bitcast(bits, jnp.float32)


def kernel(x):
    mesh = plsc.VectorSubcoreMesh(core_axis_name="c", subcore_axis_name="s")

    @functools.partial(
        pl.kernel,
        out_type=jax.ShapeDtypeStruct((B, N), jnp.float32),
        mesh=mesh,
        compiler_params=pltpu.CompilerParams(needs_layout_passes=False),
        scratch_types=[
            pltpu.VMEM((N,), jnp.float32),
            pltpu.VMEM((N,), jnp.int32),
            pltpu.VMEM((N,), jnp.int32),
            pltpu.SemaphoreType.DMA,
        ],
    )
    def run(x_hbm, out_hbm, row_v, hist_a, hist_b, sem):
        wid = lax.axis_index("s") * 2 + lax.axis_index("c")
        lane = lax.iota(jnp.int32, 16)
        ones = jnp.ones((16,), jnp.int32)
        laneoff12 = lane * NB12
        laneoff3 = lane * NB3

        def row_body(j, _):
            r = wid * ROWS + j
            cp = pltpu.async_copy(x_hbm.at[r], row_v, sem)
            _clear(hist_a, NB12 * 16)
            cp.wait()

            def p1(i, _):
                tu = _keys(row_v[pl.ds(i * 16, 16)])
                b1 = (tu >> 21).astype(jnp.int32)
                plsc.addupdate_scatter(hist_a, [b1 + laneoff12], ones)
                return 0

            lax.fori_loop(0, N // 16, p1, 0)
            b1a, base1a, b1b, base1b = _scan2(
                hist_a, NB12, jnp.int32(QA), jnp.int32(QA - 1))
            q2a = QA - base1a
            q2b = (QA - 1) - base1b

            _clear(hist_a, NB12 * 16)
            _clear(hist_b, NB12 * 16)

            def p2(i, _):
                tu = _keys(row_v[pl.ds(i * 16, 16)])
                b1 = (tu >> 21).astype(jnp.int32)
                idx = ((tu >> 10).astype(jnp.int32) & 0x7FF) + laneoff12
                plsc.addupdate_scatter(hist_a, [idx], ones, mask=b1 == b1a)
                plsc.addupdate_scatter(hist_b, [idx], ones, mask=b1 == b1b)
                return 0

            lax.fori_loop(0, N // 16, p2, 0)
            b2a, base2a = _scan1(hist_a, NB12, q2a)
            b2b, base2b = _scan1(hist_b, NB12, q2b)
            q3a = q2a - base2a
            q3b = q2b - base2b
            p21a = b1a * 2048 + b2a
            p21b = b1b * 2048 + b2b

            _clear(hist_a, NB3 * 16)
            _clear(hist_b, NB3 * 16)

            def p3(i, _):
                tu = _keys(row_v[pl.ds(i * 16, 16)])
                pref = (tu >> 10).astype(jnp.int32)
                idx = (plsc.bitcast(tu, jnp.int32) & 0x3FF) + laneoff3
                plsc.addupdate_scatter(hist_a, [idx], ones, mask=pref == p21a)
                plsc.addupdate_scatter(hist_b, [idx], ones, mask=pref == p21b)
                return 0

            lax.fori_loop(0, N // 16, p3, 0)
            b3a, _ = _scan1(hist_a, NB3, q3a)
            b3b, _ = _scan1(hist_b, NB3, q3b)

            key_a = (b1a << 21) | (b2a << 10) | b3a
            key_b = (b1b << 21) | (b2b << 10) | b3b
            va = _inv_key(jnp.full((16,), key_a, jnp.int32))
            vb = _inv_key(jnp.full((16,), key_b, jnp.int32))
            thr = (va + vb) * jnp.float32(0.5)

            def pm(i, _):
                xv = row_v[pl.ds(i * 16, 16)]
                row_v[pl.ds(i * 16, 16)] = jnp.where(
                    xv > thr, jnp.float32(1.0), jnp.float32(0.0))
                return 0

            lax.fori_loop(0, N // 16, pm, 0)
            pltpu.sync_copy(row_v, out_hbm.at[r])
            return 0

        lax.fori_loop(0, ROWS, row_body, 0)

    return run(x)


# speedup vs baseline: 19.5617x; 4.5660x over previous
import functools

import jax
import jax.numpy as jnp
from jax import lax
from jax.experimental import pallas as pl
from jax.experimental.pallas import tpu as pltpu
from jax.experimental.pallas import tpu_sc as plsc

B = 128
N = 32768
K = 1639
QA = N - K + 1
NB12 = 2048
NB3 = 1024
NW = 32
ROWS = B // NW
INT_MIN = -(2**31)
UNROLL = 8


def _keys(xv):
    b = plsc.bitcast(xv, jnp.int32)
    t = b ^ ((b >> 31) | INT_MIN)
    return plsc.bitcast(t, jnp.uint32)


def _clear(hist, nwords):
    z = jnp.zeros((16,), jnp.int32)

    @plsc.parallel_loop(0, nwords, 16, unroll=UNROLL)
    def _(i):
        hist[pl.ds(i, 16)] = z


def _scan2(hist, nb, qa, qb):
    iota = lax.iota(jnp.int32, 16)

    def body(g, carry):
        run, bin_a, base_a, bin_b, base_b = carry
        tot = hist[pl.ds(g * 16, 16)]
        for l in range(1, 16):
            tot = tot + hist[pl.ds(l * nb + g * 16, 16)]
        cum = run + plsc.cumsum(tot)
        prev = cum - tot
        ids = g * 16 + iota
        ma = (prev < qa) & (cum >= qa)
        mb = (prev < qb) & (cum >= qb)
        bin_a = bin_a + jnp.sum(jnp.where(ma, ids, 0))
        base_a = base_a + jnp.sum(jnp.where(ma, prev, 0))
        bin_b = bin_b + jnp.sum(jnp.where(mb, ids, 0))
        base_b = base_b + jnp.sum(jnp.where(mb, prev, 0))
        return run + jnp.sum(tot), bin_a, base_a, bin_b, base_b

    z = jnp.int32(0)
    _, bin_a, base_a, bin_b, base_b = lax.fori_loop(
        0, nb // 16, body, (z, z, z, z, z))
    return bin_a, base_a, bin_b, base_b


def _scan1(hist, nb, q):
    iota = lax.iota(jnp.int32, 16)

    def body(g, carry):
        run, bin_, base = carry
        tot = hist[pl.ds(g * 16, 16)]
        for l in range(1, 16):
            tot = tot + hist[pl.ds(l * nb + g * 16, 16)]
        cum = run + plsc.cumsum(tot)
        prev = cum - tot
        m = (prev < q) & (cum >= q)
        bin_ = bin_ + jnp.sum(jnp.where(m, g * 16 + iota, 0))
        base = base + jnp.sum(jnp.where(m, prev, 0))
        return run + jnp.sum(tot), bin_, base

    z = jnp.int32(0)
    _, bin_, base = lax.fori_loop(0, nb // 16, body, (z, z, z))
    return bin_, base


def _inv_key(key_i32):
    bits = key_i32 ^ jnp.where(key_i32 < 0, INT_MIN, -1)
    return plsc.bitcast(bits, jnp.float32)


def kernel(x):
    mesh = plsc.VectorSubcoreMesh(core_axis_name="c", subcore_axis_name="s")

    @functools.partial(
        pl.kernel,
        out_type=jax.ShapeDtypeStruct((B, N), jnp.float32),
        mesh=mesh,
        compiler_params=pltpu.CompilerParams(needs_layout_passes=False),
        scratch_types=[
            pltpu.VMEM((N,), jnp.float32),
            pltpu.VMEM((N,), jnp.int32),
            pltpu.VMEM((N,), jnp.int32),
            pltpu.SemaphoreType.DMA,
        ],
    )
    def run(x_hbm, out_hbm, row_v, hist_a, hist_b, sem):
        wid = lax.axis_index("s") * 2 + lax.axis_index("c")
        lane = lax.iota(jnp.int32, 16)
        ones = jnp.ones((16,), jnp.int32)
        laneoff12 = lane * NB12
        laneoff3 = lane * NB3

        def row_body(j, _):
            r = wid * ROWS + j
            cp = pltpu.async_copy(x_hbm.at[r], row_v, sem)
            _clear(hist_a, NB12 * 16)
            cp.wait()

            @plsc.parallel_loop(0, N, 16, unroll=UNROLL)
            def _(i):
                tu = _keys(row_v[pl.ds(i, 16)])
                b1 = (tu >> 21).astype(jnp.int32)
                plsc.addupdate_scatter(hist_a, [b1 + laneoff12], ones)

            b1a, base1a, b1b, base1b = _scan2(
                hist_a, NB12, jnp.int32(QA), jnp.int32(QA - 1))
            q2a = QA - base1a
            q2b = (QA - 1) - base1b

            def p2_same(_):
                _clear(hist_a, NB12 * 16)

                @plsc.parallel_loop(0, N, 16, unroll=UNROLL)
                def _(i):
                    tu = _keys(row_v[pl.ds(i, 16)])
                    b1 = (tu >> 21).astype(jnp.int32)
                    idx = ((tu >> 10).astype(jnp.int32) & 0x7FF) + laneoff12
                    plsc.addupdate_scatter(hist_a, [idx], ones, mask=b1 == b1a)

                return _scan2(hist_a, NB12, q2a, q2b)

            def p2_diff(_):
                _clear(hist_a, NB12 * 16)
                _clear(hist_b, NB12 * 16)

                @plsc.parallel_loop(0, N, 16, unroll=UNROLL)
                def _(i):
                    tu = _keys(row_v[pl.ds(i, 16)])
                    b1 = (tu >> 21).astype(jnp.int32)
                    idx = ((tu >> 10).astype(jnp.int32) & 0x7FF) + laneoff12
                    plsc.addupdate_scatter(hist_a, [idx], ones, mask=b1 == b1a)
                    plsc.addupdate_scatter(hist_b, [idx], ones, mask=b1 == b1b)

                b2a_, base2a_ = _scan1(hist_a, NB12, q2a)
                b2b_, base2b_ = _scan1(hist_b, NB12, q2b)
                return b2a_, base2a_, b2b_, base2b_

            b2a, base2a, b2b, base2b = lax.cond(b1a == b1b, p2_same, p2_diff, 0)
            q3a = q2a - base2a
            q3b = q2b - base2b
            p21a = b1a * 2048 + b2a
            p21b = b1b * 2048 + b2b

            def p3_same(_):
                _clear(hist_a, NB3 * 16)

                @plsc.parallel_loop(0, N, 16, unroll=UNROLL)
                def _(i):
                    tu = _keys(row_v[pl.ds(i, 16)])
                    pref = (tu >> 10).astype(jnp.int32)
                    idx = (plsc.bitcast(tu, jnp.int32) & 0x3FF) + laneoff3
                    plsc.addupdate_scatter(hist_a, [idx], ones, mask=pref == p21a)

                ba, _, bb, _ = _scan2(hist_a, NB3, q3a, q3b)
                return ba, bb

            def p3_diff(_):
                _clear(hist_a, NB3 * 16)
                _clear(hist_b, NB3 * 16)

                @plsc.parallel_loop(0, N, 16, unroll=UNROLL)
                def _(i):
                    tu = _keys(row_v[pl.ds(i, 16)])
                    pref = (tu >> 10).astype(jnp.int32)
                    idx = (plsc.bitcast(tu, jnp.int32) & 0x3FF) + laneoff3
                    plsc.addupdate_scatter(hist_a, [idx], ones, mask=pref == p21a)
                    plsc.addupdate_scatter(hist_b, [idx], ones, mask=pref == p21b)

                ba, _ = _scan1(hist_a, NB3, q3a)
                bb, _ = _scan1(hist_b, NB3, q3b)
                return ba, bb

            b3a, b3b = lax.cond(p21a == p21b, p3_same, p3_diff, 0)

            key_a = (b1a << 21) | (b2a << 10) | b3a
            key_b = (b1b << 21) | (b2b << 10) | b3b
            va = _inv_key(jnp.full((16,), key_a, jnp.int32))
            vb = _inv_key(jnp.full((16,), key_b, jnp.int32))
            thr = (va + vb) * 0.5

            @plsc.parallel_loop(0, N, 16, unroll=UNROLL)
            def _(i):
                xv = row_v[pl.ds(i, 16)]
                row_v[pl.ds(i, 16)] = jnp.where(xv > thr, 1.0, 0.0)

            pltpu.sync_copy(row_v, out_hbm.at[r])
            return 0

        lax.fori_loop(0, ROWS, row_body, 0)

    return run(x)


# speedup vs baseline: 20.2550x; 1.0354x over previous
import functools

import jax
import jax.numpy as jnp
from jax import lax
from jax.experimental import pallas as pl
from jax.experimental.pallas import tpu as pltpu
from jax.experimental.pallas import tpu_sc as plsc

B = 128
N = 32768
K = 1639
QA = N - K + 1
NB12 = 2048
NB3 = 1024
NW = 32
ROWS = B // NW
INT_MIN = -(2**31)
UNROLL = 8


def _clear(hist, nwords):
    z = jnp.zeros((16,), jnp.int32)

    @plsc.parallel_loop(0, nwords, 16, unroll=UNROLL)
    def _(i):
        hist[pl.ds(i, 16)] = z


def _scan2z(hist, nb, qa, qb):
    iota = lax.iota(jnp.int32, 16)
    z = jnp.zeros((16,), jnp.int32)

    def body(g, carry):
        run, bin_a, base_a, bin_b, base_b = carry
        tot = hist[pl.ds(g * 16, 16)]
        hist[pl.ds(g * 16, 16)] = z
        for l in range(1, 16):
            tot = tot + hist[pl.ds(l * nb + g * 16, 16)]
            hist[pl.ds(l * nb + g * 16, 16)] = z
        cum = run + plsc.cumsum(tot)
        prev = cum - tot
        ids = g * 16 + iota
        ma = (prev < qa) & (cum >= qa)
        mb = (prev < qb) & (cum >= qb)
        bin_a = bin_a + jnp.sum(jnp.where(ma, ids, 0))
        base_a = base_a + jnp.sum(jnp.where(ma, prev, 0))
        bin_b = bin_b + jnp.sum(jnp.where(mb, ids, 0))
        base_b = base_b + jnp.sum(jnp.where(mb, prev, 0))
        return run + jnp.sum(tot), bin_a, base_a, bin_b, base_b

    zi = jnp.int32(0)
    _, bin_a, base_a, bin_b, base_b = lax.fori_loop(
        0, nb // 16, body, (zi, zi, zi, zi, zi))
    return bin_a, base_a, bin_b, base_b


def _scan1z(hist, nb, q):
    iota = lax.iota(jnp.int32, 16)
    z = jnp.zeros((16,), jnp.int32)

    def body(g, carry):
        run, bin_, base = carry
        tot = hist[pl.ds(g * 16, 16)]
        hist[pl.ds(g * 16, 16)] = z
        for l in range(1, 16):
            tot = tot + hist[pl.ds(l * nb + g * 16, 16)]
            hist[pl.ds(l * nb + g * 16, 16)] = z
        cum = run + plsc.cumsum(tot)
        prev = cum - tot
        m = (prev < q) & (cum >= q)
        bin_ = bin_ + jnp.sum(jnp.where(m, g * 16 + iota, 0))
        base = base + jnp.sum(jnp.where(m, prev, 0))
        return run + jnp.sum(tot), bin_, base

    zi = jnp.int32(0)
    _, bin_, base = lax.fori_loop(0, nb // 16, body, (zi, zi, zi))
    return bin_, base


def _inv_key(key_i32):
    bits = key_i32 ^ jnp.where(key_i32 < 0, INT_MIN, -1)
    return plsc.bitcast(bits, jnp.float32)


def kernel(x):
    mesh = plsc.VectorSubcoreMesh(core_axis_name="c", subcore_axis_name="s")

    @functools.partial(
        pl.kernel,
        out_type=jax.ShapeDtypeStruct((B, N), jnp.float32),
        mesh=mesh,
        compiler_params=pltpu.CompilerParams(needs_layout_passes=False),
        scratch_types=[
            pltpu.VMEM((N,), jnp.float32),
            pltpu.VMEM((N,), jnp.float32),
            pltpu.VMEM((N,), jnp.int32),
            pltpu.SemaphoreType.DMA,
            pltpu.SemaphoreType.DMA,
            pltpu.SemaphoreType.DMA,
            pltpu.SemaphoreType.DMA,
        ],
    )
    def run(x_hbm, out_hbm, buf_a, buf_b, hist, sem_ia, sem_ib, sem_oa, sem_ob):
        wid = lax.axis_index("s") * 2 + lax.axis_index("c")
        lane = lax.iota(jnp.int32, 16)
        ones = jnp.ones((16,), jnp.int32)
        laneoff12 = lane * NB12
        laneoff3 = lane * NB3

        def compute_row(buf):
            @plsc.parallel_loop(0, N, 16, unroll=UNROLL)
            def _(i):
                xv = buf[pl.ds(i, 16)]
                bb = plsc.bitcast(xv, jnp.int32)
                t = bb ^ ((bb >> 31) | INT_MIN)
                buf[pl.ds(i, 16)] = plsc.bitcast(t, jnp.float32)
                tu = plsc.bitcast(t, jnp.uint32)
                b1 = (tu >> 21).astype(jnp.int32)
                plsc.addupdate_scatter(hist, [b1 + laneoff12], ones)

            b1a, base1a, b1b, base1b = _scan2z(
                hist, NB12, jnp.int32(QA), jnp.int32(QA - 1))
            q2a = QA - base1a
            q2b = (QA - 1) - base1b

            def p2_hist(b1sel):
                @plsc.parallel_loop(0, N, 16, unroll=UNROLL)
                def _(i):
                    tu = plsc.bitcast(buf[pl.ds(i, 16)], jnp.uint32)
                    b1 = (tu >> 21).astype(jnp.int32)
                    idx = ((tu >> 10).astype(jnp.int32) & 0x7FF) + laneoff12
                    plsc.addupdate_scatter(hist, [idx], ones, mask=b1 == b1sel)

            def p2_same(_):
                p2_hist(b1a)
                return _scan2z(hist, NB12, q2a, q2b)

            def p2_diff(_):
                p2_hist(b1a)
                b2a_, base2a_ = _scan1z(hist, NB12, q2a)
                p2_hist(b1b)
                b2b_, base2b_ = _scan1z(hist, NB12, q2b)
                return b2a_, base2a_, b2b_, base2b_

            b2a, base2a, b2b, base2b = lax.cond(b1a == b1b, p2_same, p2_diff, 0)
            q3a = q2a - base2a
            q3b = q2b - base2b
            p21a = b1a * 2048 + b2a
            p21b = b1b * 2048 + b2b

            def p3_hist(psel):
                @plsc.parallel_loop(0, N, 16, unroll=UNROLL)
                def _(i):
                    tu = plsc.bitcast(buf[pl.ds(i, 16)], jnp.uint32)
                    pref = (tu >> 10).astype(jnp.int32)
                    idx = (plsc.bitcast(tu, jnp.int32) & 0x3FF) + laneoff3
                    plsc.addupdate_scatter(hist, [idx], ones, mask=pref == psel)

            def p3_same(_):
                p3_hist(p21a)
                ba, _, bc, _ = _scan2z(hist, NB3, q3a, q3b)
                return ba, bc

            def p3_diff(_):
                p3_hist(p21a)
                ba, _ = _scan1z(hist, NB3, q3a)
                p3_hist(p21b)
                bc, _ = _scan1z(hist, NB3, q3b)
                return ba, bc

            b3a, b3b = lax.cond(p21a == p21b, p3_same, p3_diff, 0)

            key_a = (b1a << 21) | (b2a << 10) | b3a
            key_b = (b1b << 21) | (b2b << 10) | b3b
            va = _inv_key(jnp.full((16,), key_a, jnp.int32))
            vb = _inv_key(jnp.full((16,), key_b, jnp.int32))
            thr = (va + vb) * 0.5
            tb = plsc.bitcast(thr, jnp.int32)
            kthr = plsc.bitcast(tb ^ ((tb >> 31) | INT_MIN), jnp.uint32)

            @plsc.parallel_loop(0, N, 16, unroll=UNROLL)
            def _(i):
                tu = plsc.bitcast(buf[pl.ds(i, 16)], jnp.uint32)
                buf[pl.ds(i, 16)] = jnp.where(tu > kthr, 1.0, 0.0)

        r0 = wid * ROWS
        i_a = pltpu.async_copy(x_hbm.at[r0], buf_a, sem_ia)
        i_b = pltpu.async_copy(x_hbm.at[r0 + 1], buf_b, sem_ib)
        _clear(hist, NB12 * 16)

        i_a.wait()
        compute_row(buf_a)
        o_a = pltpu.async_copy(buf_a, out_hbm.at[r0], sem_oa)

        i_b.wait()
        o_a.wait()
        i_a = pltpu.async_copy(x_hbm.at[r0 + 2], buf_a, sem_ia)
        compute_row(buf_b)
        o_b = pltpu.async_copy(buf_b, out_hbm.at[r0 + 1], sem_ob)

        i_a.wait()
        o_b.wait()
        i_b = pltpu.async_copy(x_hbm.at[r0 + 3], buf_b, sem_ib)
        compute_row(buf_a)
        o_a = pltpu.async_copy(buf_a, out_hbm.at[r0 + 2], sem_oa)

        i_b.wait()
        compute_row(buf_b)
        o_a.wait()
        o_b = pltpu.async_copy(buf_b, out_hbm.at[r0 + 3], sem_ob)
        o_b.wait()

    return run(x)


# speedup vs baseline: 23.5023x; 1.1603x over previous
import functools

import jax
import jax.numpy as jnp
from jax import lax
from jax.experimental import pallas as pl
from jax.experimental.pallas import tpu as pltpu
from jax.experimental.pallas import tpu_sc as plsc

B = 128
N = 32768
K = 1639
QA = N - K + 1
NB1 = 2048
NW = 32
ROWS = B // NW
INT_MIN = -(2**31)
XCAP = 16384
RH_A = 16448
RH_B = RH_A + 2048


def _scan2z(hist, nb, qa, qb):
    iota = lax.iota(jnp.int32, 16)
    z = jnp.zeros((16,), jnp.int32)

    def body(g, carry):
        run, bin_a, base_a, cnt_a, bin_b, base_b, cnt_b = carry
        tot = hist[pl.ds(g * 16, 16)]
        hist[pl.ds(g * 16, 16)] = z
        for l in range(1, 16):
            tot = tot + hist[pl.ds(l * nb + g * 16, 16)]
            hist[pl.ds(l * nb + g * 16, 16)] = z
        cum = run + plsc.cumsum(tot)
        prev = cum - tot
        ids = g * 16 + iota
        ma = (prev < qa) & (cum >= qa)
        mb = (prev < qb) & (cum >= qb)
        bin_a = bin_a + jnp.sum(jnp.where(ma, ids, 0))
        base_a = base_a + jnp.sum(jnp.where(ma, prev, 0))
        cnt_a = cnt_a + jnp.sum(jnp.where(ma, tot, 0))
        bin_b = bin_b + jnp.sum(jnp.where(mb, ids, 0))
        base_b = base_b + jnp.sum(jnp.where(mb, prev, 0))
        cnt_b = cnt_b + jnp.sum(jnp.where(mb, tot, 0))
        return (run + jnp.sum(tot), bin_a, base_a, cnt_a,
                bin_b, base_b, cnt_b)

    zi = jnp.int32(0)
    _, bin_a, base_a, cnt_a, bin_b, base_b, cnt_b = lax.fori_loop(
        0, nb // 16, body, (zi, zi, zi, zi, zi, zi, zi))
    return bin_a, base_a, cnt_a, bin_b, base_b, cnt_b


def _scan1z(hist, nb, q, off):
    iota = lax.iota(jnp.int32, 16)
    z = jnp.zeros((16,), jnp.int32)

    def body(g, carry):
        run, bin_, base = carry
        tot = hist[pl.ds(off + g * 16, 16)]
        hist[pl.ds(off + g * 16, 16)] = z
        for l in range(1, 16):
            tot = tot + hist[pl.ds(off + l * nb + g * 16, 16)]
            hist[pl.ds(off + l * nb + g * 16, 16)] = z
        cum = run + plsc.cumsum(tot)
        prev = cum - tot
        m = (prev < q) & (cum >= q)
        bin_ = bin_ + jnp.sum(jnp.where(m, g * 16 + iota, 0))
        base = base + jnp.sum(jnp.where(m, prev, 0))
        return run + jnp.sum(tot), bin_, base

    zi = jnp.int32(0)
    _, bin_, base = lax.fori_loop(0, nb // 16, body, (zi, zi, zi))
    return bin_, base


def _inv_key(key_i32):
    bits = key_i32 ^ jnp.where(key_i32 < 0, INT_MIN, -1)
    return plsc.bitcast(bits, jnp.float32)


def kernel(x):
    mesh = plsc.VectorSubcoreMesh(core_axis_name="c", subcore_axis_name="s")

    @functools.partial(
        pl.kernel,
        out_type=jax.ShapeDtypeStruct((B, N), jnp.float32),
        mesh=mesh,
        compiler_params=pltpu.CompilerParams(needs_layout_passes=False),
        scratch_types=[
            pltpu.VMEM((N,), jnp.float32),
            pltpu.VMEM((N,), jnp.float32),
            pltpu.VMEM((N,), jnp.int32),
            pltpu.SemaphoreType.DMA,
            pltpu.SemaphoreType.DMA,
            pltpu.SemaphoreType.DMA,
            pltpu.SemaphoreType.DMA,
        ],
    )
    def run(x_hbm, out_hbm, buf_a, buf_b, hist, sem_ia, sem_ib, sem_oa, sem_ob):
        wid = lax.axis_index("s") * 2 + lax.axis_index("c")
        lane = lax.iota(jnp.int32, 16)
        ones = jnp.ones((16,), jnp.int32)
        zeros = jnp.zeros((16,), jnp.int32)
        laneoff1 = lane * NB1
        laneoff7 = lane * 128

        def compute_row(buf):
            @plsc.parallel_loop(0, N, 16, unroll=8)
            def _(i):
                xv = buf[pl.ds(i, 16)]
                bb = plsc.bitcast(xv, jnp.int32)
                t = bb ^ ((bb >> 31) | INT_MIN)
                buf[pl.ds(i, 16)] = plsc.bitcast(t, jnp.float32)
                tu = plsc.bitcast(t, jnp.uint32)
                b1 = (tu >> 21).astype(jnp.int32)
                plsc.addupdate_scatter(hist, [b1 + laneoff1], ones)

            b1a, base1a, cnt_a, b1b, base1b, cnt_b = _scan2z(
                hist, NB1, jnp.int32(QA), jnp.int32(QA - 1))
            q2a = QA - base1a
            q2b = (QA - 1) - base1b
            cnt = cnt_a + jnp.where(b1a == b1b, 0, cnt_b)

            def refine3(load_fn, nvec):
                qa_, qb_, pfa, pfb = q2a, q2b, b1a, b1b
                for shift in (14, 7, 0):
                    @plsc.parallel_loop(0, nvec, 1, unroll=2)
                    def _(i, _s=shift, _pa=pfa, _pb=pfb):
                        ku = load_fn(i)
                        sub = ((ku >> _s).astype(jnp.int32) & 0x7F) + laneoff7
                        pref = (ku >> (_s + 7)).astype(jnp.int32)
                        plsc.addupdate_scatter(
                            hist, [RH_A + sub], ones, mask=pref == _pa)
                        plsc.addupdate_scatter(
                            hist, [RH_B + sub], ones, mask=pref == _pb)

                    ra, ba_ = _scan1z(hist, 128, qa_, RH_A)
                    rb, bb_ = _scan1z(hist, 128, qb_, RH_B)
                    qa_ = qa_ - ba_
                    qb_ = qb_ - bb_
                    pfa = pfa * 128 + ra
                    pfb = pfb * 128 + rb
                return pfa, pfb

            def small(_):
                @plsc.parallel_loop(0, N, 16, unroll=8, carry=jnp.int32(0))
                def _ext(i, off):
                    tu = plsc.bitcast(buf[pl.ds(i, 16)], jnp.uint32)
                    b1 = (tu >> 21).astype(jnp.int32)
                    m = (b1 == b1a) | (b1 == b1b)
                    plsc.store_compressed(
                        hist.at[pl.ds(off, 16)],
                        plsc.bitcast(tu, jnp.int32), mask=m)
                    return off + jnp.sum(m.astype(jnp.int32))

                nv = (cnt + 15) >> 4
                ka, kb = refine3(
                    lambda i: plsc.bitcast(hist[pl.ds(i * 16, 16)],
                                           jnp.uint32), nv)

                @plsc.parallel_loop(0, nv * 16, 16)
                def _(i):
                    hist[pl.ds(i, 16)] = zeros

                return ka, kb

            def big(_):
                return refine3(
                    lambda i: plsc.bitcast(buf[pl.ds(i * 16, 16)],
                                           jnp.uint32), jnp.int32(N // 16))

            key_a, key_b = lax.cond(cnt <= XCAP, small, big, 0)

            va = _inv_key(jnp.full((16,), key_a, jnp.int32))
            vb = _inv_key(jnp.full((16,), key_b, jnp.int32))
            thr = (va + vb) * 0.5
            tb = plsc.bitcast(thr, jnp.int32)
            kthr = plsc.bitcast(tb ^ ((tb >> 31) | INT_MIN), jnp.uint32)

            @plsc.parallel_loop(0, N, 16, unroll=8)
            def _(i):
                tu = plsc.bitcast(buf[pl.ds(i, 16)], jnp.uint32)
                buf[pl.ds(i, 16)] = jnp.where(tu > kthr, 1.0, 0.0)

        r0 = wid * ROWS
        i_a = pltpu.async_copy(x_hbm.at[r0], buf_a, sem_ia)
        i_b = pltpu.async_copy(x_hbm.at[r0 + 1], buf_b, sem_ib)

        @plsc.parallel_loop(0, NB1 * 16, 16, unroll=8)
        def _(i):
            hist[pl.ds(i, 16)] = zeros

        i_a.wait()
        compute_row(buf_a)
        o_a = pltpu.async_copy(buf_a, out_hbm.at[r0], sem_oa)

        i_b.wait()
        o_a.wait()
        i_a = pltpu.async_copy(x_hbm.at[r0 + 2], buf_a, sem_ia)
        compute_row(buf_b)
        o_b = pltpu.async_copy(buf_b, out_hbm.at[r0 + 1], sem_ob)

        i_a.wait()
        o_b.wait()
        i_b = pltpu.async_copy(x_hbm.at[r0 + 3], buf_b, sem_ib)
        compute_row(buf_a)
        o_a = pltpu.async_copy(buf_a, out_hbm.at[r0 + 2], sem_oa)

        i_b.wait()
        compute_row(buf_b)
        o_a.wait()
        o_b = pltpu.async_copy(buf_b, out_hbm.at[r0 + 3], sem_ob)
        o_b.wait()

    return run(x)
